# baseline (speedup 1.0000x reference)
import functools

import jax
import jax.numpy as jnp
from jax import lax
from jax.experimental import pallas as pl
from jax.experimental.pallas import tpu as pltpu
from jax.experimental.pallas import tpu_sc as plsc

_N = 10000
_E = 320000
_D = 128
_H = 128
_C = 4
_G = 32

_NP = 10240
_RPT = _NP // 16
_EPT1 = _E // 32
_EPT3 = _E // 16
_K3B = 40
_CH = 4000
_NCH = _EPT3 // _CH
_CB = _CH // _K3B
_NBLK = 8
_BLK = _NP // _NBLK

_HIGH = lax.Precision.HIGHEST


@functools.cache
def _mesh():
    return plsc.VectorSubcoreMesh(core_axis_name="c", subcore_axis_name="s",
                                  num_cores=2, num_subcores=16)


def _deg_body(dst_hbm, out_hbm, dstb, degl):
    c = lax.axis_index("c")
    s = lax.axis_index("s")
    w = s * 2 + c
    pltpu.sync_copy(dst_hbm.at[pl.ds(w * _EPT1, _EPT1)], dstb)

    def zero(i, carry):
        degl[pl.ds(i * 16, 16)] = jnp.zeros((16,), jnp.float32)
        return carry
    lax.fori_loop(0, _NP // 16, zero, 0)

    ones = jnp.ones((16,), jnp.float32)

    def acc(i, carry):
        idx = dstb[pl.ds(i * 16, 16)]
        plsc.addupdate_scatter(degl, [idx], ones)
        return carry
    lax.fori_loop(0, _EPT1 // 16, acc, 0)

    pltpu.sync_copy(degl, out_hbm.at[pl.ds(w * _NP, _NP)])


def _deg_call(dst):
    k = pl.kernel(
        _deg_body,
        out_type=jax.ShapeDtypeStruct((32 * _NP,), jnp.float32),
        mesh=_mesh(),
        scratch_types=[
            pltpu.VMEM((_EPT1,), jnp.int32),
            pltpu.VMEM((_NP,), jnp.float32),
        ],
        compiler_params=pltpu.CompilerParams(needs_layout_passes=False),
    )
    return k(dst).reshape(32, _NP)


def _mm_body(x_ref, w_ref, deg_ref, hs_ref, dv_ref):
    h = jnp.dot(x_ref[...], w_ref[...], preferred_element_type=jnp.float32,
                precision=_HIGH)
    deg = jnp.sum(deg_ref[...], axis=0) + 1.0
    dinv = lax.rsqrt(deg)[:, None]
    hs_ref[...] = h * dinv
    dv_ref[...] = jnp.broadcast_to(dinv, (_BLK, _H))


def _mm_call(xp, wcat, deg):
    return pl.pallas_call(
        _mm_body,
        grid=(2, _NBLK),
        in_specs=[
            pl.BlockSpec((_BLK, _D), lambda c, i: (i, 0)),
            pl.BlockSpec((_D, _H), lambda c, i: (0, c)),
            pl.BlockSpec((32, _BLK), lambda c, i: (0, i)),
        ],
        out_specs=[
            pl.BlockSpec((_BLK, _D), lambda c, i: (c * _NBLK + i, 0)),
            pl.BlockSpec((_BLK, _H), lambda c, i: (i, 0)),
        ],
        out_shape=[
            jax.ShapeDtypeStruct((2 * _NP, _D), jnp.float32),
            jax.ShapeDtypeStruct((_NP, _H), jnp.float32),
        ],
    )(xp, wcat, deg)


def _edge_body(hs_hbm, src_hbm, dst4_hbm, out_hbm,
               srcb, dstb, rows0, rows1, sem0, sem1, ssem0, ssem1, agg):
    c = lax.axis_index("c")
    s = lax.axis_index("s")

    pltpu.sync_copy(hs_hbm.at[pl.ds(c * _NP + s * _RPT, _RPT)],
                    agg.at[pl.ds(s * _RPT, _RPT)])

    off = c * _NP

    plsc.subcore_barrier()

    def gather_start(j, rows, sem):
        pltpu.async_copy(hs_hbm.at[srcb.at[pl.ds(j * _K3B, _K3B)]], rows, sem)

    def gather_wait(j, rows, sem):
        pltpu.make_async_copy(hs_hbm.at[srcb.at[pl.ds(j * _K3B, _K3B)]],
                              rows, sem).wait()

    def chunk(sc, carry):
        pltpu.sync_copy(src_hbm.at[pl.ds(s * _EPT3 + sc * _CH, _CH)], srcb)
        pltpu.sync_copy(dst4_hbm.at[s * _NCH + sc], dstb)

        def add_off(i, carry2):
            srcb[pl.ds(i * 16, 16)] = srcb[pl.ds(i * 16, 16)] + off
            return carry2
        lax.fori_loop(0, _CH // 16, add_off, 0)

        gather_start(0, rows0, sem0)

        def step(i, carry2):
            j0 = 2 * i
            gather_start(j0 + 1, rows1, sem1)
            gather_wait(j0, rows0, sem0)
            pltpu.sync_copy(rows0, agg.at[dstb.at[j0]], add=True)

            @pl.when(i + 1 < _CB // 2)
            def _():
                gather_start(j0 + 2, rows0, sem0)
            gather_wait(j0 + 1, rows1, sem1)
            pltpu.sync_copy(rows1, agg.at[dstb.at[j0 + 1]], add=True)
            return carry2
        lax.fori_loop(0, _CB // 2, step, 0)
        return carry
    lax.fori_loop(0, _NCH, chunk, 0)

    plsc.subcore_barrier()
    pltpu.sync_copy(agg.at[pl.ds(s * _RPT, _RPT)],
                    out_hbm.at[pl.ds(c * _NP + s * _RPT, _RPT)])


def _edge_call(hs, src, dst4):
    k = pl.kernel(
        _edge_body,
        out_type=jax.ShapeDtypeStruct((2 * _NP, _D), jnp.float32),
        mesh=_mesh(),
        scratch_types=[
            pltpu.VMEM((_CH,), jnp.int32),
            pltpu.VMEM((_CB, _K3B), jnp.int32),
            pltpu.VMEM((_K3B, _D), jnp.float32),
            pltpu.VMEM((_K3B, _D), jnp.float32),
            pltpu.SemaphoreType.DMA,
            pltpu.SemaphoreType.DMA,
            pltpu.SemaphoreType.DMA,
            pltpu.SemaphoreType.DMA,
            pltpu.VMEM_SHARED((_NP, _D), jnp.float32),
        ],
    )
    return k(hs, src, dst4)


def _pool_body(a0_ref, a1_ref, dv_ref, b_ref, btd_ref, bbu_ref,
               fcw_ref, fcb_ref, out_ref, sums, cnts):
    i = pl.program_id(0)

    @pl.when(i == 0)
    def _():
        sums[...] = jnp.zeros_like(sums)
        cnts[...] = jnp.zeros_like(cnts)

    dv = dv_ref[...]
    td = a0_ref[...] * dv + btd_ref[...]
    bu = a1_ref[...] * dv + bbu_ref[...]
    feats = jnp.concatenate(
        [jnp.maximum(td, 0.0), td, jnp.maximum(bu, 0.0), bu], axis=1)
    gid = lax.broadcasted_iota(jnp.int32, (_G, 1), 0)
    m = (b_ref[0] == gid).astype(jnp.float32)
    sums[...] += lax.dot_general(m, feats, (((1,), (0,)), ((), ())),
                                 preferred_element_type=jnp.float32,
                                 precision=_HIGH)
    cnts[...] += jnp.broadcast_to(jnp.sum(m, axis=1, keepdims=True), (_G, 128))

    @pl.when(i == _NBLK - 1)
    def _():
        cnt = jnp.maximum(cnts[:, :1], 1.0)
        mean = sums[...] / cnt
        logits = jnp.dot(mean, fcw_ref[...], preferred_element_type=jnp.float32,
                         precision=_HIGH) + fcb_ref[...]
        mx = jnp.max(logits, axis=1, keepdims=True)
        lse = jnp.log(jnp.sum(jnp.exp(logits - mx), axis=1, keepdims=True)) + mx
        out_ref[...] = logits - lse


def _pool_call(agg2, dinvb, batch3, btd, bbu, fcw, fcb):
    return pl.pallas_call(
        _pool_body,
        grid=(_NBLK,),
        in_specs=[
            pl.BlockSpec((_BLK, _D), lambda i: (i, 0)),
            pl.BlockSpec((_BLK, _D), lambda i: (i + _NBLK, 0)),
            pl.BlockSpec((_BLK, _H), lambda i: (i, 0)),
            pl.BlockSpec((1, 1, _BLK), lambda i: (i, 0, 0)),
            pl.BlockSpec((1, _H), lambda i: (0, 0)),
            pl.BlockSpec((1, _H), lambda i: (0, 0)),
            pl.BlockSpec((4 * _H, _C), lambda i: (0, 0)),
            pl.BlockSpec((1, _C), lambda i: (0, 0)),
        ],
        out_specs=pl.BlockSpec((_G, _C), lambda i: (0, 0)),
        out_shape=jax.ShapeDtypeStruct((_G, _C), jnp.float32),
        scratch_shapes=[
            pltpu.VMEM((_G, 4 * _H), jnp.float32),
            pltpu.VMEM((_G, 128), jnp.float32),
        ],
    )(agg2, agg2, dinvb, batch3, btd, bbu, fcw, fcb)


def kernel(x, edge_index, batch, W_td, b_td, W_bu, b_bu, fc_W, fc_b):
    xp = jnp.zeros((_NP, _D), jnp.float32).at[:_N].set(x)
    wcat = jnp.concatenate([W_td, W_bu], axis=1)
    src = edge_index[0]
    dst = edge_index[1]
    dst4 = dst.reshape(16 * _NCH, _CB, _K3B)
    batch3 = jnp.full((_NP,), _G, jnp.int32).at[:_N].set(batch)
    batch3 = batch3.reshape(_NBLK, 1, _BLK)

    deg = _deg_call(dst)
    hs, dinvb = _mm_call(xp, wcat, deg)
    agg2 = _edge_call(hs, src, dst4)
    return _pool_call(agg2, dinvb, batch3,
                      b_td.reshape(1, _H), b_bu.reshape(1, _H),
                      fc_W, fc_b.reshape(1, _C))


# speedup vs baseline: 1.2824x; 1.2824x over previous
import functools

import jax
import jax.numpy as jnp
from jax import lax
from jax.experimental import pallas as pl
from jax.experimental.pallas import tpu as pltpu
from jax.experimental.pallas import tpu_sc as plsc

_N = 10000
_E = 320000
_D = 128
_H = 128
_C = 4
_G = 32

_NP = 10240
_RPT = _NP // 16
_EPT1 = _E // 32
_EPT3 = _E // 16
_K3B = 80
_CH = 4000
_NCH = _EPT3 // _CH
_CB = _CH // _K3B
_NBLK = 8
_BLK = _NP // _NBLK

_HIGH = lax.Precision.HIGHEST


@functools.cache
def _mesh():
    return plsc.VectorSubcoreMesh(core_axis_name="c", subcore_axis_name="s",
                                  num_cores=2, num_subcores=16)


def _deg_body(dst_hbm, out_hbm, dstb, degl):
    c = lax.axis_index("c")
    s = lax.axis_index("s")
    w = s * 2 + c
    pltpu.sync_copy(dst_hbm.at[pl.ds(w * _EPT1, _EPT1)], dstb)

    def zero(i, carry):
        degl[pl.ds(i * 16, 16)] = jnp.zeros((16,), jnp.float32)
        return carry
    lax.fori_loop(0, _NP // 16, zero, 0)

    ones = jnp.ones((16,), jnp.float32)

    def acc(i, carry):
        idx = dstb[pl.ds(i * 16, 16)]
        plsc.addupdate_scatter(degl, [idx], ones)
        return carry
    lax.fori_loop(0, _EPT1 // 16, acc, 0)

    pltpu.sync_copy(degl, out_hbm.at[pl.ds(w * _NP, _NP)])


def _deg_call(dst):
    k = pl.kernel(
        _deg_body,
        out_type=jax.ShapeDtypeStruct((32 * _NP,), jnp.float32),
        mesh=_mesh(),
        scratch_types=[
            pltpu.VMEM((_EPT1,), jnp.int32),
            pltpu.VMEM((_NP,), jnp.float32),
        ],
        compiler_params=pltpu.CompilerParams(needs_layout_passes=False),
    )
    return k(dst).reshape(32, _NP)


def _mm_body(x_ref, w_ref, deg_ref, hs_ref):
    h = jnp.dot(x_ref[...], w_ref[...], preferred_element_type=jnp.float32,
                precision=_HIGH)
    deg = jnp.sum(deg_ref[...], axis=0) + 1.0
    dinv = lax.rsqrt(deg)[:, None]
    hs_ref[...] = h * dinv


def _mm_call(xp, wcat, deg):
    return pl.pallas_call(
        _mm_body,
        grid=(2, _NBLK),
        in_specs=[
            pl.BlockSpec((_BLK, _D), lambda c, i: (i, 0)),
            pl.BlockSpec((_D, _H), lambda c, i: (0, c)),
            pl.BlockSpec((32, _BLK), lambda c, i: (0, i)),
        ],
        out_specs=pl.BlockSpec((_BLK, _D), lambda c, i: (c * _NBLK + i, 0)),
        out_shape=jax.ShapeDtypeStruct((2 * _NP, _D), jnp.float32),
    )(xp, wcat, deg)


def _edge_body(hs_hbm, src_hbm, dst4_hbm, out_hbm,
               srcb, dstb, rows0, rows1, sem0, sem1, ssem0, ssem1, agg):
    c = lax.axis_index("c")
    s = lax.axis_index("s")

    init = pltpu.async_copy(hs_hbm.at[pl.ds(c * _NP + s * _RPT, _RPT)],
                            agg.at[pl.ds(s * _RPT, _RPT)], ssem0)
    init.wait()

    plsc.subcore_barrier()

    def gather_start(j, rows, sem):
        pltpu.async_copy(hs_hbm.at[srcb.at[pl.ds(j * _K3B, _K3B)]], rows, sem)

    def gather_wait(j, rows, sem):
        pltpu.make_async_copy(hs_hbm.at[srcb.at[pl.ds(j * _K3B, _K3B)]],
                              rows, sem).wait()

    def chunk(sc, carry):
        pltpu.sync_copy(
            src_hbm.at[pl.ds(c * _E + s * _EPT3 + sc * _CH, _CH)], srcb)
        pltpu.sync_copy(dst4_hbm.at[s * _NCH + sc], dstb)

        gather_start(0, rows0, sem0)

        def step(i, carry2):
            j0 = 2 * i
            gather_start(j0 + 1, rows1, sem1)
            gather_wait(j0, rows0, sem0)
            pltpu.sync_copy(rows0, agg.at[dstb.at[j0]], add=True)

            @pl.when(i + 1 < _CB // 2)
            def _():
                gather_start(j0 + 2, rows0, sem0)
            gather_wait(j0 + 1, rows1, sem1)
            pltpu.sync_copy(rows1, agg.at[dstb.at[j0 + 1]], add=True)
            return carry2
        lax.fori_loop(0, _CB // 2, step, 0)
        return carry
    lax.fori_loop(0, _NCH, chunk, 0)

    plsc.subcore_barrier()
    pltpu.sync_copy(agg.at[pl.ds(s * _RPT, _RPT)],
                    out_hbm.at[pl.ds(c * _NP + s * _RPT, _RPT)])


def _edge_call(hs, src, dst4):
    k = pl.kernel(
        _edge_body,
        out_type=jax.ShapeDtypeStruct((2 * _NP, _D), jnp.float32),
        mesh=_mesh(),
        scratch_types=[
            pltpu.VMEM((_CH,), jnp.int32),
            pltpu.VMEM((_CB, _K3B), jnp.int32),
            pltpu.VMEM((_K3B, _D), jnp.float32),
            pltpu.VMEM((_K3B, _D), jnp.float32),
            pltpu.SemaphoreType.DMA,
            pltpu.SemaphoreType.DMA,
            pltpu.SemaphoreType.DMA,
            pltpu.SemaphoreType.DMA,
            pltpu.VMEM_SHARED((_NP, _D), jnp.float32),
        ],
    )
    return k(hs, src, dst4)


def _pool_body(a0_ref, a1_ref, deg_ref, b_ref, btd_ref, bbu_ref,
               fcw_ref, fcb_ref, out_ref, sums, cnts):
    i = pl.program_id(0)

    @pl.when(i == 0)
    def _():
        sums[...] = jnp.zeros_like(sums)
        cnts[...] = jnp.zeros_like(cnts)

    dv = lax.rsqrt(jnp.sum(deg_ref[...], axis=0) + 1.0)[:, None]
    td = a0_ref[...] * dv + btd_ref[...]
    bu = a1_ref[...] * dv + bbu_ref[...]
    feats = jnp.concatenate(
        [jnp.maximum(td, 0.0), td, jnp.maximum(bu, 0.0), bu], axis=1)
    gid = lax.broadcasted_iota(jnp.int32, (_G, 1), 0)
    m = (b_ref[0] == gid).astype(jnp.float32)
    sums[...] += lax.dot_general(m, feats, (((1,), (0,)), ((), ())),
                                 preferred_element_type=jnp.float32,
                                 precision=_HIGH)
    cnts[...] += jnp.broadcast_to(jnp.sum(m, axis=1, keepdims=True), (_G, 128))

    @pl.when(i == _NBLK - 1)
    def _():
        cnt = jnp.maximum(cnts[:, :1], 1.0)
        mean = sums[...] / cnt
        logits = jnp.dot(mean, fcw_ref[...], preferred_element_type=jnp.float32,
                         precision=_HIGH) + fcb_ref[...]
        mx = jnp.max(logits, axis=1, keepdims=True)
        lse = jnp.log(jnp.sum(jnp.exp(logits - mx), axis=1, keepdims=True)) + mx
        out_ref[...] = logits - lse


def _pool_call(agg2, deg, batch3, btd, bbu, fcw, fcb):
    return pl.pallas_call(
        _pool_body,
        grid=(_NBLK,),
        in_specs=[
            pl.BlockSpec((_BLK, _D), lambda i: (i, 0)),
            pl.BlockSpec((_BLK, _D), lambda i: (i + _NBLK, 0)),
            pl.BlockSpec((32, _BLK), lambda i: (0, i)),
            pl.BlockSpec((1, 1, _BLK), lambda i: (i, 0, 0)),
            pl.BlockSpec((1, _H), lambda i: (0, 0)),
            pl.BlockSpec((1, _H), lambda i: (0, 0)),
            pl.BlockSpec((4 * _H, _C), lambda i: (0, 0)),
            pl.BlockSpec((1, _C), lambda i: (0, 0)),
        ],
        out_specs=pl.BlockSpec((_G, _C), lambda i: (0, 0)),
        out_shape=jax.ShapeDtypeStruct((_G, _C), jnp.float32),
        scratch_shapes=[
            pltpu.VMEM((_G, 4 * _H), jnp.float32),
            pltpu.VMEM((_G, 128), jnp.float32),
        ],
    )(agg2, agg2, deg, batch3, btd, bbu, fcw, fcb)


def kernel(x, edge_index, batch, W_td, b_td, W_bu, b_bu, fc_W, fc_b):
    xp = jnp.zeros((_NP, _D), jnp.float32).at[:_N].set(x)
    wcat = jnp.concatenate([W_td, W_bu], axis=1)
    src = edge_index[0]
    dst = edge_index[1]
    src2 = jnp.concatenate([src, src + _NP])
    dst4 = dst.reshape(16 * _NCH, _CB, _K3B)
    batch3 = jnp.full((_NP,), _G, jnp.int32).at[:_N].set(batch)
    batch3 = batch3.reshape(_NBLK, 1, _BLK)

    deg = _deg_call(dst)
    hs = _mm_call(xp, wcat, deg)
    agg2 = _edge_call(hs, src2, dst4)
    return _pool_call(agg2, deg, batch3,
                      b_td.reshape(1, _H), b_bu.reshape(1, _H),
                      fc_W, fc_b.reshape(1, _C))


# speedup vs baseline: 1.2853x; 1.0023x over previous
import functools

import jax
import jax.numpy as jnp
from jax import lax
from jax.experimental import pallas as pl
from jax.experimental.pallas import tpu as pltpu
from jax.experimental.pallas import tpu_sc as plsc

_N = 10000
_E = 320000
_D = 128
_H = 128
_C = 4
_G = 32

_NP = 10240
_RPT = _NP // 16
_EPT1 = _E // 32
_EPT3 = _E // 16
_K3B = 80
_CH = 4000
_NCH = _EPT3 // _CH
_CB = _CH // _K3B
_NBLK = 8
_BLK = _NP // _NBLK

_HIGH = lax.Precision.HIGHEST


@functools.cache
def _mesh():
    return plsc.VectorSubcoreMesh(core_axis_name="c", subcore_axis_name="s",
                                  num_cores=2, num_subcores=16)


def _deg_body(dst_hbm, out_hbm, dstb, degl):
    c = lax.axis_index("c")
    s = lax.axis_index("s")
    w = s * 2 + c
    pltpu.sync_copy(dst_hbm.at[pl.ds(w * _EPT1, _EPT1)], dstb)

    def zero(i, carry):
        degl[pl.ds(i * 16, 16)] = jnp.zeros((16,), jnp.float32)
        return carry
    lax.fori_loop(0, _NP // 16, zero, 0)

    ones = jnp.ones((16,), jnp.float32)

    def acc(i, carry):
        idx = dstb[pl.ds(i * 16, 16)]
        plsc.addupdate_scatter(degl, [idx], ones)
        return carry
    lax.fori_loop(0, _EPT1 // 16, acc, 0)

    pltpu.sync_copy(degl, out_hbm.at[pl.ds(w * _NP, _NP)])


def _deg_call(dst):
    k = pl.kernel(
        _deg_body,
        out_type=jax.ShapeDtypeStruct((32 * _NP,), jnp.float32),
        mesh=_mesh(),
        scratch_types=[
            pltpu.VMEM((_EPT1,), jnp.int32),
            pltpu.VMEM((_NP,), jnp.float32),
        ],
        compiler_params=pltpu.CompilerParams(needs_layout_passes=False),
    )
    return k(dst).reshape(32, _NP)


def _mm_body(x_ref, w_ref, deg_ref, hs_ref):
    h = jnp.dot(x_ref[...], w_ref[...], preferred_element_type=jnp.float32,
                precision=_HIGH)
    deg = jnp.sum(deg_ref[...], axis=0) + 1.0
    dinv = lax.rsqrt(deg)[:, None]
    hs_ref[...] = h * dinv


def _mm_call(xp, wcat, deg):
    return pl.pallas_call(
        _mm_body,
        grid=(_NBLK, 2),
        in_specs=[
            pl.BlockSpec((_BLK, _D), lambda i, c: (i, 0)),
            pl.BlockSpec((_D, _H), lambda i, c: (0, c)),
            pl.BlockSpec((32, _BLK), lambda i, c: (0, i)),
        ],
        out_specs=pl.BlockSpec((_BLK, _D), lambda i, c: (c * _NBLK + i, 0)),
        out_shape=jax.ShapeDtypeStruct((2 * _NP, _D), jnp.float32),
    )(xp, wcat, deg)


def _edge_body(hs_hbm, src_hbm, dst4_hbm, out_hbm,
               srcb, dstb, rows0, rows1, sem0, sem1, ssem0, ssem1, agg):
    c = lax.axis_index("c")
    s = lax.axis_index("s")

    init = pltpu.async_copy(hs_hbm.at[pl.ds(c * _NP + s * _RPT, _RPT)],
                            agg.at[pl.ds(s * _RPT, _RPT)], ssem0)
    init.wait()

    plsc.subcore_barrier()

    def gather_start(j, rows, sem):
        pltpu.async_copy(hs_hbm.at[srcb.at[pl.ds(j * _K3B, _K3B)]], rows, sem)

    def gather_wait(j, rows, sem):
        pltpu.make_async_copy(hs_hbm.at[srcb.at[pl.ds(j * _K3B, _K3B)]],
                              rows, sem).wait()

    def chunk(sc, carry):
        pltpu.sync_copy(
            src_hbm.at[pl.ds(c * _E + s * _EPT3 + sc * _CH, _CH)], srcb)
        pltpu.sync_copy(dst4_hbm.at[s * _NCH + sc], dstb)

        gather_start(0, rows0, sem0)

        def step(i, carry2):
            j0 = 2 * i
            gather_start(j0 + 1, rows1, sem1)
            gather_wait(j0, rows0, sem0)
            pltpu.sync_copy(rows0, agg.at[dstb.at[j0]], add=True)

            @pl.when(i + 1 < _CB // 2)
            def _():
                gather_start(j0 + 2, rows0, sem0)
            gather_wait(j0 + 1, rows1, sem1)
            pltpu.sync_copy(rows1, agg.at[dstb.at[j0 + 1]], add=True)
            return carry2
        lax.fori_loop(0, _CB // 2, step, 0)
        return carry
    lax.fori_loop(0, _NCH, chunk, 0)

    plsc.subcore_barrier()
    pltpu.sync_copy(agg.at[pl.ds(s * _RPT, _RPT)],
                    out_hbm.at[pl.ds(c * _NP + s * _RPT, _RPT)])


def _edge_call(hs, src, dst4):
    k = pl.kernel(
        _edge_body,
        out_type=jax.ShapeDtypeStruct((2 * _NP, _D), jnp.float32),
        mesh=_mesh(),
        scratch_types=[
            pltpu.VMEM((_CH,), jnp.int32),
            pltpu.VMEM((_CB, _K3B), jnp.int32),
            pltpu.VMEM((_K3B, _D), jnp.float32),
            pltpu.VMEM((_K3B, _D), jnp.float32),
            pltpu.SemaphoreType.DMA,
            pltpu.SemaphoreType.DMA,
            pltpu.SemaphoreType.DMA,
            pltpu.SemaphoreType.DMA,
            pltpu.VMEM_SHARED((_NP, _D), jnp.float32),
        ],
    )
    return k(hs, src, dst4)


def _pool_body(a0_ref, a1_ref, deg_ref, b_ref, btd_ref, bbu_ref,
               fcw_ref, fcb_ref, out_ref, sums, cnts):
    i = pl.program_id(0)

    @pl.when(i == 0)
    def _():
        sums[...] = jnp.zeros_like(sums)
        cnts[...] = jnp.zeros_like(cnts)

    dv = lax.rsqrt(jnp.sum(deg_ref[...], axis=0) + 1.0)[:, None]
    td = a0_ref[...] * dv + btd_ref[...]
    bu = a1_ref[...] * dv + bbu_ref[...]
    feats = jnp.concatenate(
        [jnp.maximum(td, 0.0), td, jnp.maximum(bu, 0.0), bu], axis=1)
    gid = lax.broadcasted_iota(jnp.int32, (_G, 1), 0)
    m = (b_ref[0] == gid).astype(jnp.float32)
    sums[...] += lax.dot_general(m, feats, (((1,), (0,)), ((), ())),
                                 preferred_element_type=jnp.float32,
                                 precision=_HIGH)
    cnts[...] += jnp.broadcast_to(jnp.sum(m, axis=1, keepdims=True), (_G, 128))

    @pl.when(i == _NBLK - 1)
    def _():
        cnt = jnp.maximum(cnts[:, :1], 1.0)
        mean = sums[...] / cnt
        logits = jnp.dot(mean, fcw_ref[...], preferred_element_type=jnp.float32,
                         precision=_HIGH) + fcb_ref[...]
        mx = jnp.max(logits, axis=1, keepdims=True)
        lse = jnp.log(jnp.sum(jnp.exp(logits - mx), axis=1, keepdims=True)) + mx
        out_ref[...] = logits - lse


def _pool_call(agg2, deg, batch3, btd, bbu, fcw, fcb):
    return pl.pallas_call(
        _pool_body,
        grid=(_NBLK,),
        in_specs=[
            pl.BlockSpec((_BLK, _D), lambda i: (i, 0)),
            pl.BlockSpec((_BLK, _D), lambda i: (i + _NBLK, 0)),
            pl.BlockSpec((32, _BLK), lambda i: (0, i)),
            pl.BlockSpec((1, 1, _BLK), lambda i: (i, 0, 0)),
            pl.BlockSpec((1, _H), lambda i: (0, 0)),
            pl.BlockSpec((1, _H), lambda i: (0, 0)),
            pl.BlockSpec((4 * _H, _C), lambda i: (0, 0)),
            pl.BlockSpec((1, _C), lambda i: (0, 0)),
        ],
        out_specs=pl.BlockSpec((_G, _C), lambda i: (0, 0)),
        out_shape=jax.ShapeDtypeStruct((_G, _C), jnp.float32),
        scratch_shapes=[
            pltpu.VMEM((_G, 4 * _H), jnp.float32),
            pltpu.VMEM((_G, 128), jnp.float32),
        ],
    )(agg2, agg2, deg, batch3, btd, bbu, fcw, fcb)


def kernel(x, edge_index, batch, W_td, b_td, W_bu, b_bu, fc_W, fc_b):
    xp = jnp.zeros((_NP, _D), jnp.float32).at[:_N].set(x)
    wcat = jnp.concatenate([W_td, W_bu], axis=1)
    src = edge_index[0]
    dst = edge_index[1]
    src2 = jnp.concatenate([src, src + _NP])
    dst4 = dst.reshape(16 * _NCH, _CB, _K3B)
    batch3 = jnp.full((_NP,), _G, jnp.int32).at[:_N].set(batch)
    batch3 = batch3.reshape(_NBLK, 1, _BLK)

    deg = _deg_call(dst)
    hs = _mm_call(xp, wcat, deg)
    agg2 = _edge_call(hs, src2, dst4)
    return _pool_call(agg2, deg, batch3,
                      b_td.reshape(1, _H), b_bu.reshape(1, _H),
                      fc_W, fc_b.reshape(1, _C))
